# baseline (speedup 1.0000x reference)
import functools

import jax
import jax.numpy as jnp
from jax import lax
from jax.experimental import pallas as pl
from jax.experimental.pallas import tpu as pltpu
from jax.experimental.pallas import tpu_sc as plsc

_CHUNK = 128
_NBUF = 8


def _build_lookup(NW, NCHUNK, CHUNK, V, D, NC):
    N = NW * NCHUNK * CHUNK
    mesh = plsc.VectorSubcoreMesh(core_axis_name="c", subcore_axis_name="s")

    @functools.partial(
        pl.kernel,
        out_type=jax.ShapeDtypeStruct((N, D), jnp.float32),
        mesh=mesh,
        scratch_types=[
            pltpu.VMEM((NCHUNK, CHUNK), jnp.int32),
            pltpu.VMEM((_NBUF, CHUNK, D), jnp.float32),
            pltpu.SemaphoreType.DMA((_NBUF,)),
            pltpu.SemaphoreType.DMA((_NBUF,)),
        ],
        compiler_params=pltpu.CompilerParams(use_tc_tiling_on_sc=False),
    )
    def emb_kernel(table_hbm, idx_hbm, out_hbm, idx_v, rows_v, gsem, osem):
        wid = lax.axis_index("s") * NC + lax.axis_index("c")
        base = wid * (NCHUNK * CHUNK)
        pltpu.sync_copy(idx_hbm.at[wid], idx_v)

        @pl.loop(0, NCHUNK, step=_NBUF)
        def _chunk_group(c0):
            gathers = [
                pltpu.async_copy(
                    table_hbm.at[idx_v.at[c0 + b]], rows_v.at[b], gsem.at[b]
                )
                for b in range(_NBUF)
            ]
            writes = []
            for b in range(_NBUF):
                gathers[b].wait()
                writes.append(
                    pltpu.async_copy(
                        rows_v.at[b],
                        out_hbm.at[pl.ds(base + (c0 + b) * CHUNK, CHUNK)],
                        osem.at[b],
                    )
                )
            for w in writes:
                w.wait()

    return emb_kernel


def kernel(sentence, table):
    B, S = sentence.shape
    V, D = table.shape
    N = B * S

    info = plsc.get_sparse_core_info()
    NC, NS = info.num_cores, info.num_subcores
    NW = NC * NS
    assert N % (NW * _CHUNK) == 0
    NCHUNK = N // (NW * _CHUNK)
    assert NCHUNK % _NBUF == 0

    idx = sentence.reshape(NW, NCHUNK, _CHUNK).astype(jnp.int32)
    out = _build_lookup(NW, NCHUNK, _CHUNK, V, D, NC)(table, idx)
    return out.reshape(B, S, D)
